# baseline (speedup 1.0000x reference)
import functools

import jax
import jax.numpy as jnp
from jax import lax
from jax.experimental import pallas as pl
from jax.experimental.pallas import tpu as pltpu
from jax.experimental.pallas import tpu_sc as plsc

N = 10000
DEG = 32
D = 128

P_TC = 5376
BLK_TC = 384
L = 16
NB = 8
NW = 32
N_SC = N - P_TC
NBLK = N_SC // NB
TMAX = (NBLK + NW - 1) // NW
TMAX += TMAX % 2
NDC = D // L

_mesh = plsc.VectorSubcoreMesh(core_axis_name="c", subcore_axis_name="s")


_QB, _RB = divmod(NBLK, NW)
_WLEN = (_QB + 1) * NB


def _gat_sc_body(a1_hbm, a2_hbm, ft_hbm, out_hbm,
                 ft_buf, a2w_buf, a1w_buf, out_buf,
                 in_sem0, in_sem1, out_sem0, out_sem1):
    wid = lax.axis_index("c") * 16 + lax.axis_index("s")
    in_sems = (in_sem0, in_sem1)
    out_sems = (out_sem0, out_sem1)

    nblk_w = jnp.where(wid < _RB, _QB + 1, _QB)
    wstart = P_TC + (wid * _QB + jnp.minimum(wid, _RB)) * NB
    nbase = jnp.minimum(wstart, N - _WLEN)
    delta = wstart - nbase
    pltpu.sync_copy(a1_hbm.at[pl.ds(nbase, _WLEN)], a1w_buf)
    pltpu.sync_copy(a2_hbm.at[:, pl.ds(nbase, _WLEN)], a2w_buf)

    iota0 = lax.iota(jnp.int32, L)
    iota1 = iota0 + L

    def start_in(t, b):
        @pl.when(t < nblk_w)
        def _():
            pltpu.make_async_copy(
                ft_hbm.at[pl.ds(wstart + t * NB, NB)], ft_buf.at[b],
                in_sems[b]).start()

    def compute_block(t, b):
        def node_body(i, _):
            off = jnp.full((L,), delta + t * NB + i, jnp.int32)
            a1v = plsc.load_gather(a1w_buf, [off])
            x0 = plsc.load_gather(a2w_buf, [iota0, off]) + a1v
            x1 = plsc.load_gather(a2w_buf, [iota1, off]) + a1v
            l0 = jnp.where(x0 > 0, x0, x0 * 0.01)
            l1 = jnp.where(x1 > 0, x1, x1 * 0.01)
            m = jnp.maximum(jnp.max(l0), jnp.max(l1))
            e0 = jnp.exp(l0 - m)
            e1 = jnp.exp(l1 - m)
            sv = jnp.zeros((L,), jnp.float32) + (jnp.sum(e0) + jnp.sum(e1))
            e0 = e0 / sv
            e1 = e1 / sv

            acc = [jnp.zeros((L,), jnp.float32) for _ in range(NDC)]
            for k in range(DEG):
                w = e0[k] if k < L else e1[k - L]
                for dc in range(NDC):
                    acc[dc] = acc[dc] + w * ft_buf[b, i, k, pl.ds(dc * L, L)]
            for dc in range(NDC):
                out_buf[b, i, pl.ds(dc * L, L)] = acc[dc]
            return 0

        lax.fori_loop(0, NB, node_body, 0)

    start_in(0, 0)
    start_in(1, 1)

    def ring_body(tt, _):
        for b in (0, 1):
            t = tt * 2 + b

            @pl.when(t < nblk_w)
            def _():
                pltpu.make_async_copy(
                    ft_hbm.at[pl.ds(wstart + t * NB, NB)], ft_buf.at[b],
                    in_sems[b]).wait()

                @pl.when(t >= 2)
                def _():
                    pltpu.make_async_copy(
                        out_buf.at[b],
                        out_hbm.at[pl.ds(wstart + (t - 2) * NB, NB)],
                        out_sems[b]).wait()

                compute_block(t, b)
                pltpu.make_async_copy(
                    out_buf.at[b], out_hbm.at[pl.ds(wstart + t * NB, NB)],
                    out_sems[b]).start()
                start_in(t + 2, b)
        return 0

    lax.fori_loop(0, (_QB + 2) // 2, ring_body, 0)

    for b in (0, 1):
        pltpu.make_async_copy(out_buf.at[b], out_hbm.at[pl.ds(0, NB)],
                              out_sems[b]).wait()


_gat_sc = functools.partial(
    pl.kernel,
    out_type=jax.ShapeDtypeStruct((N, D), jnp.float32),
    mesh=_mesh,
    compiler_params=pltpu.CompilerParams(needs_layout_passes=False,
                                         use_tc_tiling_on_sc=False),
    scratch_types=[
        pltpu.VMEM((2, NB, DEG, D), jnp.float32),
        pltpu.VMEM((DEG, _WLEN), jnp.float32),
        pltpu.VMEM((_WLEN,), jnp.float32),
        pltpu.VMEM((2, NB, D), jnp.float32),
        pltpu.SemaphoreType.DMA,
        pltpu.SemaphoreType.DMA,
        pltpu.SemaphoreType.DMA,
        pltpu.SemaphoreType.DMA,
    ],
)(_gat_sc_body)


def _gat_tc_block(a1_ref, a2_ref, ft_ref, out_ref):
    i = pl.program_id(0)
    a1 = a1_ref[pl.ds(i * BLK_TC, BLK_TC)]
    a2 = a2_ref[...]
    ft = ft_ref[...]
    a = a1[None, :] + a2
    l = jnp.where(a > 0, a, 0.01 * a)
    m = jnp.max(l, axis=0, keepdims=True)
    e = jnp.exp(l - m)
    wk = e / jnp.sum(e, axis=0, keepdims=True)
    w = wk.T
    out_ref[...] = jnp.sum(w[:, :, None] * ft, axis=1)


def _gat_tc(a1f, a2km, ft):
    return pl.pallas_call(
        _gat_tc_block,
        grid=(P_TC // BLK_TC,),
        in_specs=[
            pl.BlockSpec((N,), lambda i: (0,)),
            pl.BlockSpec((DEG, BLK_TC), lambda i: (0, i)),
            pl.BlockSpec((BLK_TC, DEG, D), lambda i: (i, 0, 0)),
        ],
        out_specs=pl.BlockSpec((BLK_TC, D), lambda i: (i, 0)),
        out_shape=jax.ShapeDtypeStruct((P_TC, D), jnp.float32),
    )(a1f, a2km, ft)


@jax.jit
def kernel(a1, a2, ft):
    a1f = a1.reshape(N)
    a2km = a2.reshape(N, DEG).T
    out_sc = _gat_sc(a1f, a2km, ft)
    out_tc = _gat_tc(a1f, a2km, ft)
    return lax.dynamic_update_slice(out_sc, out_tc, (0, 0))


# speedup vs baseline: 1.0356x; 1.0356x over previous
import functools

import jax
import jax.numpy as jnp
from jax import lax
from jax.experimental import pallas as pl
from jax.experimental.pallas import tpu as pltpu
from jax.experimental.pallas import tpu_sc as plsc

N = 10000
DEG = 32
D = 128

P_TC = 5632
BLK_TC = 512
L = 16
NB = 8
NW = 32
N_SC = N - P_TC
NBLK = N_SC // NB
TMAX = (NBLK + NW - 1) // NW
TMAX += TMAX % 2
NDC = D // L

_mesh = plsc.VectorSubcoreMesh(core_axis_name="c", subcore_axis_name="s")


_QB, _RB = divmod(NBLK, NW)
_WLEN = (_QB + 1) * NB


def _gat_sc_body(a1_hbm, a2_hbm, ft_hbm, out_hbm,
                 ft_buf, a2w_buf, a1w_buf, out_buf,
                 in_sem0, in_sem1, out_sem0, out_sem1):
    wid = lax.axis_index("c") * 16 + lax.axis_index("s")
    in_sems = (in_sem0, in_sem1)
    out_sems = (out_sem0, out_sem1)

    nblk_w = jnp.where(wid < _RB, _QB + 1, _QB)
    wstart = P_TC + (wid * _QB + jnp.minimum(wid, _RB)) * NB
    nbase = jnp.minimum(wstart, N - _WLEN)
    delta = wstart - nbase
    pltpu.sync_copy(a1_hbm.at[pl.ds(nbase, _WLEN)], a1w_buf)
    pltpu.sync_copy(a2_hbm.at[:, pl.ds(nbase, _WLEN)], a2w_buf)

    iota0 = lax.iota(jnp.int32, L)
    iota1 = iota0 + L

    def start_in(t, b):
        @pl.when(t < nblk_w)
        def _():
            pltpu.make_async_copy(
                ft_hbm.at[pl.ds(wstart + t * NB, NB)], ft_buf.at[b],
                in_sems[b]).start()

    def compute_block(t, b):
        def node_body(i, _):
            off = jnp.full((L,), delta + t * NB + i, jnp.int32)
            a1v = plsc.load_gather(a1w_buf, [off])
            x0 = plsc.load_gather(a2w_buf, [iota0, off]) + a1v
            x1 = plsc.load_gather(a2w_buf, [iota1, off]) + a1v
            l0 = jnp.where(x0 > 0, x0, x0 * 0.01)
            l1 = jnp.where(x1 > 0, x1, x1 * 0.01)
            m = jnp.maximum(jnp.max(l0), jnp.max(l1))
            e0 = jnp.exp(l0 - m)
            e1 = jnp.exp(l1 - m)
            sv = jnp.zeros((L,), jnp.float32) + (jnp.sum(e0) + jnp.sum(e1))
            e0 = e0 / sv
            e1 = e1 / sv

            acc = [jnp.zeros((L,), jnp.float32) for _ in range(NDC)]
            for k in range(DEG):
                w = e0[k] if k < L else e1[k - L]
                for dc in range(NDC):
                    acc[dc] = acc[dc] + w * ft_buf[b, i, k, pl.ds(dc * L, L)]
            for dc in range(NDC):
                out_buf[b, i, pl.ds(dc * L, L)] = acc[dc]
            return 0

        lax.fori_loop(0, NB, node_body, 0)

    start_in(0, 0)
    start_in(1, 1)

    def ring_body(tt, _):
        for b in (0, 1):
            t = tt * 2 + b

            @pl.when(t < nblk_w)
            def _():
                pltpu.make_async_copy(
                    ft_hbm.at[pl.ds(wstart + t * NB, NB)], ft_buf.at[b],
                    in_sems[b]).wait()

                @pl.when(t >= 2)
                def _():
                    pltpu.make_async_copy(
                        out_buf.at[b],
                        out_hbm.at[pl.ds(wstart + (t - 2) * NB, NB)],
                        out_sems[b]).wait()

                compute_block(t, b)
                pltpu.make_async_copy(
                    out_buf.at[b], out_hbm.at[pl.ds(wstart + t * NB, NB)],
                    out_sems[b]).start()
                start_in(t + 2, b)
        return 0

    lax.fori_loop(0, (_QB + 2) // 2, ring_body, 0)

    for b in (0, 1):
        pltpu.make_async_copy(out_buf.at[b], out_hbm.at[pl.ds(0, NB)],
                              out_sems[b]).wait()


_gat_sc = functools.partial(
    pl.kernel,
    out_type=jax.ShapeDtypeStruct((N, D), jnp.float32),
    mesh=_mesh,
    compiler_params=pltpu.CompilerParams(needs_layout_passes=False,
                                         use_tc_tiling_on_sc=False),
    scratch_types=[
        pltpu.VMEM((2, NB, DEG, D), jnp.float32),
        pltpu.VMEM((DEG, _WLEN), jnp.float32),
        pltpu.VMEM((_WLEN,), jnp.float32),
        pltpu.VMEM((2, NB, D), jnp.float32),
        pltpu.SemaphoreType.DMA,
        pltpu.SemaphoreType.DMA,
        pltpu.SemaphoreType.DMA,
        pltpu.SemaphoreType.DMA,
    ],
)(_gat_sc_body)


def _gat_tc_block(a1_ref, a2_ref, ft_ref, out_ref):
    i = pl.program_id(0)
    a1 = a1_ref[pl.ds(i * BLK_TC, BLK_TC)]
    a2 = a2_ref[...]
    ft = ft_ref[...]
    a = a1[None, :] + a2
    l = jnp.where(a > 0, a, 0.01 * a)
    m = jnp.max(l, axis=0, keepdims=True)
    e = jnp.exp(l - m)
    wk = e / jnp.sum(e, axis=0, keepdims=True)
    w = wk.T
    out_ref[...] = jnp.sum(w[:, :, None] * ft, axis=1)


def _gat_tc(a1f, a2km, ft):
    return pl.pallas_call(
        _gat_tc_block,
        grid=(P_TC // BLK_TC,),
        in_specs=[
            pl.BlockSpec((N,), lambda i: (0,)),
            pl.BlockSpec((DEG, BLK_TC), lambda i: (0, i)),
            pl.BlockSpec((BLK_TC, DEG, D), lambda i: (i, 0, 0)),
        ],
        out_specs=pl.BlockSpec((BLK_TC, D), lambda i: (i, 0)),
        out_shape=jax.ShapeDtypeStruct((P_TC, D), jnp.float32),
    )(a1f, a2km, ft)


@jax.jit
def kernel(a1, a2, ft):
    a1f = a1.reshape(N)
    a2km = a2.reshape(N, DEG).T
    out_sc = _gat_sc(a1f, a2km, ft)
    out_tc = _gat_tc(a1f, a2km, ft)
    return lax.dynamic_update_slice(out_sc, out_tc, (0, 0))
